# baseline (speedup 1.0000x reference)
import jax
import jax.numpy as jnp
from jax.experimental import pallas as pl

_BLOCK_ROWS = 4000


def _matmul_block(x_ref, w_ref, o_ref):
    o_ref[...] = jnp.dot(x_ref[...], w_ref[...],
                         preferred_element_type=jnp.float32)


def kernel(atomic_numbers, atomic_energies):
    n, k = atomic_numbers.shape
    m = atomic_energies.shape[1]
    grid = n // _BLOCK_ROWS
    return pl.pallas_call(
        _matmul_block,
        grid=(grid,),
        in_specs=[
            pl.BlockSpec((_BLOCK_ROWS, k), lambda i: (i, 0)),
            pl.BlockSpec((k, m), lambda i: (0, 0)),
        ],
        out_specs=pl.BlockSpec((_BLOCK_ROWS, m), lambda i: (i, 0)),
        out_shape=jax.ShapeDtypeStruct((n, m), jnp.float32),
    )(atomic_numbers, atomic_energies)


# speedup vs baseline: 1.0625x; 1.0625x over previous
import jax
import jax.numpy as jnp
from jax.experimental import pallas as pl

_BLOCK_ROWS = 20000


def _matmul_block(x_ref, w_ref, o_ref):
    o_ref[...] = jnp.dot(x_ref[...], w_ref[...],
                         preferred_element_type=jnp.float32)


def kernel(atomic_numbers, atomic_energies):
    n, k = atomic_numbers.shape
    m = atomic_energies.shape[1]
    grid = n // _BLOCK_ROWS
    return pl.pallas_call(
        _matmul_block,
        grid=(grid,),
        in_specs=[
            pl.BlockSpec((_BLOCK_ROWS, k), lambda i: (i, 0)),
            pl.BlockSpec((k, m), lambda i: (0, 0)),
        ],
        out_specs=pl.BlockSpec((_BLOCK_ROWS, m), lambda i: (i, 0)),
        out_shape=jax.ShapeDtypeStruct((n, m), jnp.float32),
    )(atomic_numbers, atomic_energies)
